# baseline (speedup 1.0000x reference)
import functools

import jax
import jax.numpy as jnp
from jax import lax
from jax.experimental import pallas as pl
from jax.experimental.pallas import tpu as pltpu
from jax.experimental.pallas import tpu_sc as plsc

NC = 2
NS = 16
LANES = 16
CHUNK = 50


def _feat_matmul(user_feat, item_feat, u_w, v_w):
    n_users, d = user_feat.shape
    n_items = item_feat.shape[0]
    n = n_users + n_items
    br = 1000
    r_blocks = n_users // br

    def body(uf_ref, if_ref, uw_ref, vw_ref, out_ref):
        p = pl.program_id(0)
        x = jnp.where(p == 0, uf_ref[...], if_ref[...])
        w = jnp.where(p == 0, uw_ref[...], vw_ref[...])
        out_ref[...] = jnp.dot(x, w, preferred_element_type=jnp.float32,
                               precision=lax.Precision.HIGHEST)

    return pl.pallas_call(
        body,
        grid=(2, r_blocks),
        in_specs=[
            pl.BlockSpec((br, d), lambda p, r: (r, 0)),
            pl.BlockSpec((br, d), lambda p, r: (r, 0)),
            pl.BlockSpec((d, d), lambda p, r: (0, 0)),
            pl.BlockSpec((d, d), lambda p, r: (0, 0)),
        ],
        out_specs=pl.BlockSpec((br, d), lambda p, r: (p * r_blocks + r, 0)),
        out_shape=jax.ShapeDtypeStruct((n, d), jnp.float32),
    )(user_feat, item_feat, u_w, v_w)


def _make_spmm(n, d, e):
    epc = e // NC
    ept = epc // NS
    nch = ept // CHUNK
    BCH = 8
    nblk = nch // BCH
    assert nch % BCH == 0
    n_pad = -(-n // (NS * LANES)) * (NS * LANES)
    nr = n_pad // NS
    zr = 16
    mesh = plsc.VectorSubcoreMesh(core_axis_name="c", subcore_axis_name="s")

    @functools.partial(
        pl.kernel,
        out_type=jax.ShapeDtypeStruct((NC, n_pad, d), jnp.float32),
        mesh=mesh,
        scratch_types=[
            pltpu.VMEM((2, BCH, CHUNK), jnp.int32),
            pltpu.VMEM((2, BCH, CHUNK), jnp.int32),
            pltpu.VMEM((2, BCH, CHUNK), jnp.float32),
            pltpu.VMEM((CHUNK, d), jnp.float32),
            pltpu.VMEM((CHUNK, d), jnp.float32),
            pltpu.VMEM((CHUNK, d), jnp.float32),
            pltpu.VMEM((CHUNK, d), jnp.float32),
            pltpu.VMEM((zr, d), jnp.float32),
            pltpu.VMEM_SHARED((n_pad, d), jnp.float32),
            pltpu.SemaphoreType.DMA,
            pltpu.SemaphoreType.DMA,
            pltpu.SemaphoreType.DMA,
            pltpu.SemaphoreType.DMA,
            pltpu.SemaphoreType.DMA,
            pltpu.SemaphoreType.DMA,
            pltpu.SemaphoreType.DMA,
            pltpu.SemaphoreType.DMA,
            pltpu.SemaphoreType.DMA,
        ],
    )
    def spmm(feat_ref, rows_ref, cols_ref, vals_ref, out_ref,
             colv3, rowv3, valv3, g0, g1, g2, g3, zbuf, acc,
             sg0, sg1, sg2, sg3, ss0, ss1, ss2, ss3, si):
        c = lax.axis_index("c")
        s = lax.axis_index("s")
        cb = (c * NS + s) * nch

        def cidx(ref3, i):
            return ref3.at[lax.rem(i // BCH, 2), lax.rem(i, BCH)]

        i0 = pltpu.async_copy(cols_ref.at[pl.ds(cb, BCH)], colv3.at[0], si)
        i1 = pltpu.async_copy(rows_ref.at[pl.ds(cb, BCH)], rowv3.at[0], si)
        i2 = pltpu.async_copy(vals_ref.at[pl.ds(cb, BCH)], valv3.at[0], si)

        for t in range(zr):
            for j in range(d // LANES):
                zbuf[t, pl.ds(j * LANES, LANES)] = jnp.zeros((LANES,), jnp.float32)
        for kb in range(0, nr // zr, 10):
            zdesc = []
            for k in range(kb, min(kb + 10, nr // zr)):
                zdesc.append(pltpu.async_copy(
                    zbuf, acc.at[pl.ds(s * nr + k * zr, zr)], sg0))
            for dsc in zdesc:
                dsc.wait()
        i0.wait()
        i1.wait()
        i2.wait()
        plsc.subcore_barrier()

        bufs = ((g0, sg0, ss0), (g1, sg1, ss1), (g2, sg2, ss2), (g3, sg3, ss3))
        RING = 4

        def step(i, k):
            gb, sg, ss = bufs[k]
            blk = i // BCH
            ib = lax.rem(i, BCH)

            @pl.when(i >= RING - 1)
            def _():
                ko = (k + 1) % RING
                pltpu.make_async_copy(bufs[ko][0],
                                      acc.at[cidx(rowv3, i + 1 - RING)],
                                      bufs[ko][2]).wait()

            @pl.when(i + 1 < nch)
            def _():
                pltpu.async_copy(feat_ref.at[cidx(colv3, i + 1)],
                                 bufs[(k + 1) % RING][0],
                                 bufs[(k + 1) % RING][1])

            @pl.when(jnp.logical_and(ib == RING, blk + 1 < nblk))
            def _():
                par = lax.rem(blk + 1, 2)
                off = cb + (blk + 1) * BCH
                pltpu.async_copy(cols_ref.at[pl.ds(off, BCH)], colv3.at[par], si)
                pltpu.async_copy(rows_ref.at[pl.ds(off, BCH)], rowv3.at[par], si)
                pltpu.async_copy(vals_ref.at[pl.ds(off, BCH)], valv3.at[par], si)

            @pl.when(jnp.logical_and(ib == BCH - 1, blk + 1 < nblk))
            def _():
                par = lax.rem(blk + 1, 2)
                off = cb + (blk + 1) * BCH
                pltpu.make_async_copy(cols_ref.at[pl.ds(off, BCH)], colv3.at[par], si).wait()
                pltpu.make_async_copy(rows_ref.at[pl.ds(off, BCH)], rowv3.at[par], si).wait()
                pltpu.make_async_copy(vals_ref.at[pl.ds(off, BCH)], valv3.at[par], si).wait()

            pltpu.make_async_copy(feat_ref.at[cidx(colv3, i)], gb, sg).wait()

            par_i = lax.rem(i // BCH, 2)

            def scale16(g, carry2):
                vv = valv3[par_i, ib, pl.ds(g * LANES, LANES)]
                for t in range(LANES):
                    v = vv[t]
                    e_i = g * LANES + t
                    for j in range(d // LANES):
                        sl = pl.ds(j * LANES, LANES)
                        gb[e_i, sl] = gb[e_i, sl] * v
                return carry2
            lax.fori_loop(0, CHUNK // LANES, scale16, 0)
            if CHUNK % LANES:
                vv = valv3[par_i, ib, pl.ds(CHUNK - LANES, LANES)]
                for t in range(LANES - CHUNK % LANES, LANES):
                    v = vv[t]
                    e_i = CHUNK - LANES + t
                    for j in range(d // LANES):
                        sl = pl.ds(j * LANES, LANES)
                        gb[e_i, sl] = gb[e_i, sl] * v
            pltpu.async_copy(gb, acc.at[cidx(rowv3, i)], ss, add=True)

        pltpu.async_copy(feat_ref.at[colv3.at[0, 0]], g0, sg0)

        def quad(j, carry):
            for k in range(RING):
                step(RING * j + k, k)
            return carry
        lax.fori_loop(0, nch // RING, quad, 0)

        for i in range(nch - RING + 1, nch):
            k = i % RING
            pltpu.make_async_copy(bufs[k][0], acc.at[cidx(rowv3, i)],
                                  bufs[k][2]).wait()
        plsc.subcore_barrier()

        r0 = s * nr
        pltpu.sync_copy(acc.at[pl.ds(r0, nr)], out_ref.at[c, pl.ds(r0, nr)])

    return spmm, n_pad


def _combine_relu(partials, n, d):
    br = 1000
    r_blocks = n // br

    def body(p_ref, out_ref):
        out_ref[...] = jnp.maximum(p_ref[0] + p_ref[1], 0.0)

    return pl.pallas_call(
        body,
        grid=(r_blocks,),
        in_specs=[pl.BlockSpec((2, br, d), lambda r: (0, r, 0))],
        out_specs=pl.BlockSpec((br, d), lambda r: (r, 0)),
        out_shape=jax.ShapeDtypeStruct((n, d), jnp.float32),
    )(partials)


def kernel(adj_vals, user_feat, item_feat, u_w, v_w, edge_index):
    n_users, d = user_feat.shape
    n = n_users + item_feat.shape[0]
    e = adj_vals.shape[0]

    feat = _feat_matmul(user_feat, item_feat, u_w, v_w)
    spmm, _ = _make_spmm(n, d, e)
    rows2 = edge_index[0].reshape(-1, CHUNK)
    cols2 = edge_index[1].reshape(-1, CHUNK)
    vals2 = adj_vals.reshape(-1, CHUNK)
    partials = spmm(feat, rows2, cols2, vals2)
    return _combine_relu(partials, n, d)


# speedup vs baseline: 1.1191x; 1.1191x over previous
import functools

import jax
import jax.numpy as jnp
from jax import lax
from jax.experimental import pallas as pl
from jax.experimental.pallas import tpu as pltpu
from jax.experimental.pallas import tpu_sc as plsc

NC = 2
NS = 16
LANES = 16
CHUNK = 125


def _feat_matmul(user_feat, item_feat, u_w, v_w):
    n_users, d = user_feat.shape
    n_items = item_feat.shape[0]
    n = n_users + n_items
    br = 1000
    r_blocks = n_users // br

    def body(uf_ref, if_ref, uw_ref, vw_ref, out_ref):
        p = pl.program_id(0)
        x = jnp.where(p == 0, uf_ref[...], if_ref[...])
        w = jnp.where(p == 0, uw_ref[...], vw_ref[...])
        out_ref[...] = jnp.dot(x, w, preferred_element_type=jnp.float32,
                               precision=lax.Precision.HIGHEST)

    return pl.pallas_call(
        body,
        grid=(2, r_blocks),
        in_specs=[
            pl.BlockSpec((br, d), lambda p, r: (r, 0)),
            pl.BlockSpec((br, d), lambda p, r: (r, 0)),
            pl.BlockSpec((d, d), lambda p, r: (0, 0)),
            pl.BlockSpec((d, d), lambda p, r: (0, 0)),
        ],
        out_specs=pl.BlockSpec((br, d), lambda p, r: (p * r_blocks + r, 0)),
        out_shape=jax.ShapeDtypeStruct((n, d), jnp.float32),
    )(user_feat, item_feat, u_w, v_w)


def _make_spmm(n, d, e):
    epc = e // NC
    ept = epc // NS
    nch = ept // CHUNK
    BCH = 16
    nblk = nch // BCH
    assert nch % BCH == 0
    n_pad = -(-n // (NS * LANES)) * (NS * LANES)
    nr = n_pad // NS
    zr = 16
    mesh = plsc.VectorSubcoreMesh(core_axis_name="c", subcore_axis_name="s")

    @functools.partial(
        pl.kernel,
        out_type=jax.ShapeDtypeStruct((NC, n_pad, d), jnp.float32),
        mesh=mesh,
        scratch_types=[
            pltpu.VMEM((2, BCH, CHUNK), jnp.int32),
            pltpu.VMEM((2, BCH, CHUNK), jnp.int32),
            pltpu.VMEM((2, BCH, CHUNK), jnp.float32),
            pltpu.VMEM((CHUNK, d), jnp.float32),
            pltpu.VMEM((CHUNK, d), jnp.float32),
            pltpu.VMEM((zr, d), jnp.float32),
            pltpu.VMEM_SHARED((n_pad, d), jnp.float32),
            pltpu.SemaphoreType.DMA,
            pltpu.SemaphoreType.DMA,
            pltpu.SemaphoreType.DMA,
            pltpu.SemaphoreType.DMA,
            pltpu.SemaphoreType.DMA,
        ],
    )
    def spmm(feat_ref, rows_ref, cols_ref, vals_ref, out_ref,
             colv3, rowv3, valv3, g0, g1, zbuf, acc, sg0, sg1, ss0, ss1, si):
        c = lax.axis_index("c")
        s = lax.axis_index("s")
        cb = (c * NS + s) * nch

        def cidx(ref3, i):
            return ref3.at[lax.rem(i // BCH, 2), lax.rem(i, BCH)]

        i0 = pltpu.async_copy(cols_ref.at[pl.ds(cb, BCH)], colv3.at[0], si)
        i1 = pltpu.async_copy(rows_ref.at[pl.ds(cb, BCH)], rowv3.at[0], si)
        i2 = pltpu.async_copy(vals_ref.at[pl.ds(cb, BCH)], valv3.at[0], si)

        for t in range(zr):
            for j in range(d // LANES):
                zbuf[t, pl.ds(j * LANES, LANES)] = jnp.zeros((LANES,), jnp.float32)
        for kb in range(0, nr // zr, 10):
            zdesc = []
            for k in range(kb, min(kb + 10, nr // zr)):
                zdesc.append(pltpu.async_copy(
                    zbuf, acc.at[pl.ds(s * nr + k * zr, zr)], sg0))
            for dsc in zdesc:
                dsc.wait()
        i0.wait()
        i1.wait()
        i2.wait()
        plsc.subcore_barrier()

        bufs = ((g0, sg0, ss0), (g1, sg1, ss1))

        def step(i, k):
            gb, sg, ss = bufs[k]
            gbo, sgo, sso = bufs[1 - k]
            blk = i // BCH
            ib = lax.rem(i, BCH)

            @pl.when(i >= 1)
            def _():
                pltpu.make_async_copy(gbo, acc.at[cidx(rowv3, i - 1)], sso).wait()

            @pl.when(jnp.logical_and(ib == 2, blk + 1 < nblk))
            def _():
                par = lax.rem(blk + 1, 2)
                off = cb + (blk + 1) * BCH
                pltpu.async_copy(cols_ref.at[pl.ds(off, BCH)], colv3.at[par], si)
                pltpu.async_copy(rows_ref.at[pl.ds(off, BCH)], rowv3.at[par], si)
                pltpu.async_copy(vals_ref.at[pl.ds(off, BCH)], valv3.at[par], si)

            @pl.when(jnp.logical_and(ib == BCH - 1, blk + 1 < nblk))
            def _():
                par = lax.rem(blk + 1, 2)
                off = cb + (blk + 1) * BCH
                pltpu.make_async_copy(cols_ref.at[pl.ds(off, BCH)], colv3.at[par], si).wait()
                pltpu.make_async_copy(rows_ref.at[pl.ds(off, BCH)], rowv3.at[par], si).wait()
                pltpu.make_async_copy(vals_ref.at[pl.ds(off, BCH)], valv3.at[par], si).wait()

            @pl.when(i + 1 < nch)
            def _():
                pltpu.async_copy(feat_ref.at[cidx(colv3, i + 1)], gbo, sgo)

            pltpu.make_async_copy(feat_ref.at[cidx(colv3, i)], gb, sg).wait()

            par_i = lax.rem(i // BCH, 2)

            def scale16(g, carry2):
                vv = valv3[par_i, ib, pl.ds(g * LANES, LANES)]
                for t in range(LANES):
                    v = vv[t]
                    e_i = g * LANES + t
                    for j in range(d // LANES):
                        sl = pl.ds(j * LANES, LANES)
                        gb[e_i, sl] = gb[e_i, sl] * v
                return carry2
            lax.fori_loop(0, CHUNK // LANES, scale16, 0)
            if CHUNK % LANES:
                vv = valv3[par_i, ib, pl.ds(CHUNK - LANES, LANES)]
                for t in range(LANES - CHUNK % LANES, LANES):
                    v = vv[t]
                    e_i = CHUNK - LANES + t
                    for j in range(d // LANES):
                        sl = pl.ds(j * LANES, LANES)
                        gb[e_i, sl] = gb[e_i, sl] * v
            pltpu.async_copy(gb, acc.at[cidx(rowv3, i)], ss, add=True)

        pltpu.async_copy(feat_ref.at[colv3.at[0, 0]], g0, sg0)

        def pair(j, carry):
            step(2 * j, 0)
            step(2 * j + 1, 1)
            return carry
        lax.fori_loop(0, nch // 2, pair, 0)

        lastk = (nch - 1) % 2
        pltpu.make_async_copy(bufs[lastk][0], acc.at[cidx(rowv3, nch - 1)],
                              bufs[lastk][2]).wait()
        plsc.subcore_barrier()

        r0 = s * nr
        pltpu.sync_copy(acc.at[pl.ds(r0, nr)], out_ref.at[c, pl.ds(r0, nr)])

    return spmm, n_pad


def _combine_relu(partials, n, d):
    br = 1000
    r_blocks = n // br

    def body(p_ref, out_ref):
        out_ref[...] = jnp.maximum(p_ref[0] + p_ref[1], 0.0)

    return pl.pallas_call(
        body,
        grid=(r_blocks,),
        in_specs=[pl.BlockSpec((2, br, d), lambda r: (0, r, 0))],
        out_specs=pl.BlockSpec((br, d), lambda r: (r, 0)),
        out_shape=jax.ShapeDtypeStruct((n, d), jnp.float32),
    )(partials)


def kernel(adj_vals, user_feat, item_feat, u_w, v_w, edge_index):
    n_users, d = user_feat.shape
    n = n_users + item_feat.shape[0]
    e = adj_vals.shape[0]

    feat = _feat_matmul(user_feat, item_feat, u_w, v_w)
    spmm, _ = _make_spmm(n, d, e)
    rows2 = edge_index[0].reshape(-1, CHUNK)
    cols2 = edge_index[1].reshape(-1, CHUNK)
    vals2 = adj_vals.reshape(-1, CHUNK)
    partials = spmm(feat, rows2, cols2, vals2)
    return jnp.maximum(partials[0, :n] + partials[1, :n], 0.0)
